# baseline (speedup 1.0000x reference)
import math

import jax
import jax.numpy as jnp
from jax.experimental import pallas as pl
from jax.experimental.pallas import tpu as pltpu

_B, _N, _C = 4, 4096, 32
_DPHI, _H1, _H2 = 2048, 128, 128
_R, _K, _MPHY = 8, 1024, 4
_J = _R * _MPHY
_BETA = 0.25
_BN = 1024
_BNH = _BN // 2
_NBLK = _N // _BN
_QINV = 1.0 / math.sqrt(10.0 + 1e-9)
_LOSS_SCALE = (1.0 + _BETA) / float(_B * _N * _C * _C)
_DN = (((1,), (0,)), ((), ()))


def _body(z_ref, phi_ref, w1_ref, b1_ref, w2_ref, b2_ref, w3_ref, b3_ref,
          cbs_ref, logits_ref, probs_ref, modes_ref, idx_ref, zq_ref,
          loss_ref, sym_ref, cb_scr, esq_scr, sele_scr, selo_scr):
    f32 = jnp.float32
    b = pl.program_id(0)
    j = pl.program_id(1)

    @pl.when(jnp.logical_and(b == 0, j == 0))
    def _():
        loss_ref[...] = jnp.zeros((1, 1), f32)
        row = jax.lax.broadcasted_iota(jnp.int32, (_BN, _BNH), 0)
        pair = jax.lax.broadcasted_iota(jnp.int32, (_BN, _BNH), 1)
        sele_scr[...] = (row == 2 * pair).astype(f32)
        selo_scr[...] = (row == 2 * pair + 1).astype(f32)

    @pl.when(j == 0)
    def _():
        ph = phi_ref[pl.ds(b, 1), :]
        h = jnp.maximum(
            jax.lax.dot_general(ph, w1_ref[...], _DN,
                                preferred_element_type=f32) + b1_ref[...],
            0.0)
        h = jnp.maximum(
            jax.lax.dot_general(h, w2_ref[...], _DN,
                                preferred_element_type=f32) + b2_ref[...],
            0.0)
        logits = jax.lax.dot_general(h, w3_ref[...], _DN,
                                     preferred_element_type=f32) + b3_ref[...]
        logits_ref[pl.ds(b, 1), :] = logits
        mx = jnp.max(logits, axis=-1, keepdims=True)
        ex = jnp.exp(logits - mx)
        probs_ref[pl.ds(b, 1), :] = ex / jnp.sum(ex, axis=-1, keepdims=True)
        lane = jax.lax.broadcasted_iota(jnp.int32, (1, _J), 1)
        mode = jnp.min(jnp.where(logits == mx, lane, _J), axis=-1,
                       keepdims=True)
        expert = mode // _MPHY
        modes_ref[pl.ds(b, 1), 0:1] = mode
        modes_ref[pl.ds(b, 1), 1:2] = expert
        modes_ref[pl.ds(b, 1), 2:3] = mode - _MPHY * expert
        modes_ref[pl.ds(b, 1), 3:4] = mode
        acc = cbs_ref[0] * (expert == 0).astype(f32)
        for r in range(1, _R):
            acc = acc + cbs_ref[r] * (expert == r).astype(f32)
        cb_scr[...] = acc
        esq_scr[...] = jnp.sum(acc * acc, axis=-1, keepdims=True)

    zb = z_ref[0]
    cb = cb_scr[...]
    crossT = jax.lax.dot_general(cb, zb, (((1,), (1,)), ((), ())),
                                 preferred_element_type=f32)
    z_sq_row = jnp.sum(zb * zb, axis=-1, keepdims=True).T
    dT = z_sq_row + esq_scr[...] - 2.0 * crossT
    idx_row = jnp.argmin(dT, axis=0)[None, :]
    kiota = jax.lax.broadcasted_iota(jnp.int32, (_K, _BN), 0)
    onehotT = (kiota == idx_row).astype(f32)
    zq_blk = jax.lax.dot_general(onehotT, cb, (((0,), (0,)), ((), ())),
                                 preferred_element_type=f32)
    diff = zq_blk - zb
    loss_ref[...] += jnp.sum(diff * diff).reshape(1, 1)

    @pl.when(jnp.logical_and(b == _B - 1, j == _NBLK - 1))
    def _():
        loss_ref[...] = loss_ref[...] * _LOSS_SCALE

    idx_ref[pl.ds(b, 1), pl.ds(j * _BN, _BN)] = idx_row
    zq_ref[0] = zb + (zq_blk - zb)

    idxf = idx_row.astype(f32)
    i0 = jax.lax.dot_general(idxf, sele_scr[...], _DN,
                             preferred_element_type=f32).astype(jnp.int32)
    i1 = jax.lax.dot_general(idxf, selo_scr[...], _DN,
                             preferred_element_type=f32).astype(jnp.int32)
    s_list = [
        i0 >> 6,
        (i0 >> 2) & 15,
        ((i0 & 3) << 2) | (i1 >> 8),
        (i1 >> 4) & 15,
        i1 & 15,
    ]
    for jj in range(5):
        s = s_list[jj]
        x = ((s >> 2) * 2 - 3).astype(f32) * _QINV
        y = ((s & 3) * 2 - 3).astype(f32) * _QINV
        sym_ref[0, jj:jj + 1, 0:1, :] = x.reshape(1, 1, _BNH)
        sym_ref[0, jj:jj + 1, 1:2, :] = y.reshape(1, 1, _BNH)


def kernel(z_e, phi, W1, b1, W2, b2, W3, b3, codebooks):
    f32 = jnp.float32
    i32 = jnp.int32
    outs = pl.pallas_call(
        _body,
        grid=(_B, _NBLK),
        in_specs=[
            pl.BlockSpec((1, _BN, _C), lambda b, j: (b, j, 0)),
            pl.BlockSpec((_B, _DPHI), lambda b, j: (0, 0)),
            pl.BlockSpec((_DPHI, _H1), lambda b, j: (0, 0)),
            pl.BlockSpec((1, _H1), lambda b, j: (0, 0)),
            pl.BlockSpec((_H1, _H2), lambda b, j: (0, 0)),
            pl.BlockSpec((1, _H2), lambda b, j: (0, 0)),
            pl.BlockSpec((_H2, _J), lambda b, j: (0, 0)),
            pl.BlockSpec((1, _J), lambda b, j: (0, 0)),
            pl.BlockSpec((_R, _K, _C), lambda b, j: (0, 0, 0)),
        ],
        out_specs=[
            pl.BlockSpec((_B, _J), lambda b, j: (0, 0)),
            pl.BlockSpec((_B, _J), lambda b, j: (0, 0)),
            pl.BlockSpec((_B, 4), lambda b, j: (0, 0)),
            pl.BlockSpec((_B, _N), lambda b, j: (0, 0)),
            pl.BlockSpec((1, _BN, _C), lambda b, j: (b, j, 0)),
            pl.BlockSpec((1, 1), lambda b, j: (0, 0)),
            pl.BlockSpec((1, 5, 2, _BNH), lambda b, j: (b, 0, 0, j)),
        ],
        out_shape=[
            jax.ShapeDtypeStruct((_B, _J), f32),
            jax.ShapeDtypeStruct((_B, _J), f32),
            jax.ShapeDtypeStruct((_B, 4), i32),
            jax.ShapeDtypeStruct((_B, _N), i32),
            jax.ShapeDtypeStruct((_B, _N, _C), f32),
            jax.ShapeDtypeStruct((1, 1), f32),
            jax.ShapeDtypeStruct((_B, 5, 2, _N // 2), f32),
        ],
        scratch_shapes=[
            pltpu.VMEM((_K, _C), f32),
            pltpu.VMEM((_K, 1), f32),
            pltpu.VMEM((_BN, _BNH), f32),
            pltpu.VMEM((_BN, _BNH), f32),
        ],
    )(z_e, phi, W1, b1.reshape(1, _H1), W2, b2.reshape(1, _H2), W3,
      b3.reshape(1, _J), codebooks)
    logits, probs, modes, indices, z_q_st, loss, sym = outs

    symbols = sym.transpose(0, 3, 1, 2).reshape(_B, _N * 10 // 4, 2)
    return (z_q_st, indices, loss[0, 0], logits, probs, modes[:, 0],
            modes[:, 2], symbols)
